# baseline (speedup 1.0000x reference)
import functools

import jax
import jax.numpy as jnp
from jax.experimental import pallas as pl
from jax.experimental.pallas import tpu as pltpu

D_MODEL = 4096
D_HID = 2048
NUM_EXPERTS = 64
TOP_K = 8
NUM_TOKENS = 16384

BT = 1024
BH = 512
GI = NUM_TOKENS // BT
GJ = D_HID // BH

_SIGN_LOW = 0x7FFFFFFF
_IDX_MASK = NUM_EXPERTS - 1
_VAL_MASK = -NUM_EXPERTS
_KEY_MIN = -(2 ** 31)


def _to_key(f):
    s = jax.lax.bitcast_convert_type(f, jnp.int32)
    return s ^ (jax.lax.shift_right_arithmetic(s, 31) & _SIGN_LOW)


def _from_key(k):
    return jax.lax.bitcast_convert_type(
        k ^ (jax.lax.shift_right_arithmetic(k, 31) & _SIGN_LOW), jnp.float32)


def _gating_body(x_ref, w1_ref, b1_ref, w2_ref, b2_ref,
                 gates_ref, idx_ref, loss_ref,
                 xbf_ref, acc_ref, esum_ref):
    i = pl.program_id(0)
    j = pl.program_id(1)

    @pl.when(j == 0)
    def _():
        xbf_ref[...] = x_ref[...].astype(jnp.bfloat16)

    h = jnp.dot(xbf_ref[...], w1_ref[...],
                preferred_element_type=jnp.float32)
    h = jnp.maximum(h + b1_ref[...], 0.0)
    part = jnp.dot(h.astype(jnp.bfloat16), w2_ref[...],
                   preferred_element_type=jnp.float32)

    @pl.when(j == 0)
    def _():
        acc_ref[...] = part

    @pl.when(j > 0)
    def _():
        acc_ref[...] += part

    @pl.when(j == GJ - 1)
    def _():
        logits = acc_ref[...] + b2_ref[...]
        iota = jax.lax.broadcasted_iota(jnp.int32, (BT, NUM_EXPERTS), 1)
        key = (_to_key(logits) & _VAL_MASK) | (_IDX_MASK - iota)
        top_keys = []
        for _k in range(TOP_K):
            m = jnp.max(key, axis=1, keepdims=True)
            top_keys.append(m)
            key = jnp.where(key == m, _KEY_MIN, key)
        tk = jnp.concatenate(top_keys, axis=1)
        ti = _IDX_MASK - (tk & _IDX_MASK)
        tv = _from_key(tk & _VAL_MASK)
        row_max = tv[:, :1]
        e = jnp.exp(tv - row_max)
        gates_ref[...] = e / jnp.sum(e, axis=1, keepdims=True)
        idx_ref[...] = ti

        pe = jnp.exp(logits - row_max)
        probs = pe / jnp.sum(pe, axis=1, keepdims=True)
        psum = jnp.sum(probs, axis=0, keepdims=True)

        @pl.when(i == 0)
        def _():
            esum_ref[...] = psum

        @pl.when(i > 0)
        def _():
            esum_ref[...] += psum

        @pl.when(i == GI - 1)
        def _():
            expert_probs = esum_ref[...] * (1.0 / NUM_TOKENS)
            log_input = jnp.log(expert_probs + 1e-08)
            target = 1.0 / NUM_EXPERTS
            loss_ref[...] = jnp.sum(target * (jnp.log(target) - log_input),
                                    keepdims=True)


@functools.partial(jax.jit, static_argnames=("interpret",))
def _gating(x, w1, b1, w2, b2, interpret=False):
    grid = (GI, GJ)
    out = pl.pallas_call(
        _gating_body,
        grid=grid,
        in_specs=[
            pl.BlockSpec((BT, D_MODEL), lambda i, j: (i, 0)),
            pl.BlockSpec((D_MODEL, BH), lambda i, j: (0, j)),
            pl.BlockSpec((1, BH), lambda i, j: (0, j)),
            pl.BlockSpec((BH, NUM_EXPERTS), lambda i, j: (j, 0)),
            pl.BlockSpec((1, NUM_EXPERTS), lambda i, j: (0, 0)),
        ],
        out_specs=[
            pl.BlockSpec((BT, TOP_K), lambda i, j: (i, 0)),
            pl.BlockSpec((BT, TOP_K), lambda i, j: (i, 0)),
            pl.BlockSpec((1, 1), lambda i, j: (0, 0)),
        ],
        out_shape=[
            jax.ShapeDtypeStruct((NUM_TOKENS, TOP_K), jnp.float32),
            jax.ShapeDtypeStruct((NUM_TOKENS, TOP_K), jnp.int32),
            jax.ShapeDtypeStruct((1, 1), jnp.float32),
        ],
        scratch_shapes=[
            pltpu.VMEM((BT, D_MODEL), jnp.bfloat16),
            pltpu.VMEM((BT, NUM_EXPERTS), jnp.float32),
            pltpu.VMEM((1, NUM_EXPERTS), jnp.float32),
        ],
        interpret=interpret,
    )(x, w1, b1, w2, b2)
    return out


def kernel(x, training, W1, b1, W2, b2, interpret=False):
    del training
    gates, idx, loss = _gating(x, W1.astype(jnp.bfloat16),
                               b1.reshape(1, D_HID),
                               W2.astype(jnp.bfloat16),
                               b2.reshape(1, NUM_EXPERTS),
                               interpret=interpret)
    return gates, idx, loss.reshape(())


# speedup vs baseline: 1.0592x; 1.0592x over previous
import functools

import jax
import jax.numpy as jnp
from jax.experimental import pallas as pl
from jax.experimental.pallas import tpu as pltpu

D_MODEL = 4096
D_HID = 2048
NUM_EXPERTS = 64
TOP_K = 8
NUM_TOKENS = 16384

BT = 1024
BH = 512
GI = NUM_TOKENS // BT
GJ = D_HID // BH

_SIGN_LOW = 0x7FFFFFFF
_IDX_MASK = NUM_EXPERTS - 1
_VAL_MASK = -NUM_EXPERTS
_KEY_MIN = -(2 ** 31)


def _to_key(f):
    s = jax.lax.bitcast_convert_type(f, jnp.int32)
    return s ^ (jax.lax.shift_right_arithmetic(s, 31) & _SIGN_LOW)


def _from_key(k):
    return jax.lax.bitcast_convert_type(
        k ^ (jax.lax.shift_right_arithmetic(k, 31) & _SIGN_LOW), jnp.float32)


def _gating_body(x_ref, w1_ref, b1_ref, w2_ref, b2_ref,
                 gates_ref, idx_ref, loss_ref,
                 acc_ref, esum_ref):
    i = pl.program_id(0)
    j = pl.program_id(1)

    h = jnp.dot(x_ref[...], w1_ref[...],
                preferred_element_type=jnp.float32)
    h = jnp.maximum(h + b1_ref[...], 0.0)
    part = jnp.dot(h, w2_ref[...],
                   preferred_element_type=jnp.float32)

    @pl.when(j == 0)
    def _():
        acc_ref[...] = part

    @pl.when(j > 0)
    def _():
        acc_ref[...] += part

    @pl.when(j == GJ - 1)
    def _():
        logits = acc_ref[...] + b2_ref[...]
        iota = jax.lax.broadcasted_iota(jnp.int32, (BT, NUM_EXPERTS), 1)
        key = (_to_key(logits) & _VAL_MASK) | (_IDX_MASK - iota)
        top_keys = []
        for _k in range(TOP_K):
            m = jnp.max(key, axis=1, keepdims=True)
            top_keys.append(m)
            key = jnp.where(key == m, _KEY_MIN, key)
        tk = jnp.concatenate(top_keys, axis=1)
        ti = _IDX_MASK - (tk & _IDX_MASK)
        tv = _from_key(tk & _VAL_MASK)
        row_max = tv[:, :1]
        e = jnp.exp(tv - row_max)
        gates_ref[...] = e / jnp.sum(e, axis=1, keepdims=True)
        idx_ref[...] = ti

        pe = jnp.exp(logits - row_max)
        probs = pe / jnp.sum(pe, axis=1, keepdims=True)
        psum = jnp.sum(probs, axis=0, keepdims=True)

        @pl.when(i == 0)
        def _():
            esum_ref[...] = psum

        @pl.when(i > 0)
        def _():
            esum_ref[...] += psum

        @pl.when(i == GI - 1)
        def _():
            expert_probs = esum_ref[...] * (1.0 / NUM_TOKENS)
            log_input = jnp.log(expert_probs + 1e-08)
            target = 1.0 / NUM_EXPERTS
            loss_ref[...] = jnp.sum(target * (jnp.log(target) - log_input),
                                    keepdims=True)


@functools.partial(jax.jit, static_argnames=("interpret",))
def _gating(x, w1, b1, w2, b2, interpret=False):
    grid = (GI, GJ)
    out = pl.pallas_call(
        _gating_body,
        grid=grid,
        in_specs=[
            pl.BlockSpec((BT, D_MODEL), lambda i, j: (i, 0)),
            pl.BlockSpec((D_MODEL, BH), lambda i, j: (0, j)),
            pl.BlockSpec((1, BH), lambda i, j: (0, j)),
            pl.BlockSpec((BH, NUM_EXPERTS), lambda i, j: (j, 0)),
            pl.BlockSpec((1, NUM_EXPERTS), lambda i, j: (0, 0)),
        ],
        out_specs=[
            pl.BlockSpec((BT, TOP_K), lambda i, j: (i, 0)),
            pl.BlockSpec((BT, TOP_K), lambda i, j: (i, 0)),
            pl.BlockSpec((1, 1), lambda i, j: (0, 0)),
        ],
        out_shape=[
            jax.ShapeDtypeStruct((NUM_TOKENS, TOP_K), jnp.float32),
            jax.ShapeDtypeStruct((NUM_TOKENS, TOP_K), jnp.int32),
            jax.ShapeDtypeStruct((1, 1), jnp.float32),
        ],
        scratch_shapes=[
            pltpu.VMEM((BT, NUM_EXPERTS), jnp.float32),
            pltpu.VMEM((1, NUM_EXPERTS), jnp.float32),
        ],
        interpret=interpret,
    )(x, w1, b1, w2, b2)
    return out


def kernel(x, training, W1, b1, W2, b2, interpret=False):
    del training
    gates, idx, loss = _gating(x, W1, b1.reshape(1, D_HID),
                               W2, b2.reshape(1, NUM_EXPERTS),
                               interpret=interpret)
    return gates, idx, loss.reshape(())
